# baseline (speedup 1.0000x reference)
import functools

import jax
import jax.numpy as jnp
from jax import lax
from jax.experimental import pallas as pl
from jax.experimental.pallas import tpu as pltpu
from jax.experimental.pallas import tpu_sc as plsc

_NC = 2
_NS = 16
_NW = _NC * _NS


@functools.cache
def _build_lookup(n_batch: int, hist: int, d: int):
  chunk = n_batch // _NW
  n_chunks = hist
  assert chunk % 8 == 0 and n_chunks % 2 == 0 and n_chunks >= 4

  mesh = plsc.VectorSubcoreMesh(
      core_axis_name="c", subcore_axis_name="s",
      num_cores=_NC, num_subcores=_NS)

  @functools.partial(
      pl.kernel,
      mesh=mesh,
      out_type=jax.ShapeDtypeStruct((hist * n_batch, d), jnp.float32),
      compiler_params=pltpu.CompilerParams(use_tc_tiling_on_sc=False),
      scratch_types=(
          [pltpu.VMEM((2, chunk), jnp.int32),
           pltpu.VMEM((2, chunk, d), jnp.float32)]
          + [pltpu.SemaphoreType.DMA] * 6
      ),
  )
  def lookup(xt_hbm, w_hbm, out_hbm, idx_v, rows_v, *sems):
    isems = sems[0:2]
    gsems = sems[2:4]
    osems = sems[4:6]
    wid = lax.axis_index("s") * _NC + lax.axis_index("c")
    col0 = wid * chunk

    def idx_load(c, b):
      return pltpu.make_async_copy(
          xt_hbm.at[c, pl.ds(col0, chunk)], idx_v.at[b], isems[b])

    def gather(c, b):
      return pltpu.make_async_copy(
          w_hbm.at[idx_v.at[b]], rows_v.at[b], gsems[b])

    def put(c, b):
      return pltpu.make_async_copy(
          rows_v.at[b],
          out_hbm.at[pl.ds(c * n_batch + col0, chunk)], osems[b])

    idx_load(0, 0).start()
    idx_load(1, 1).start()
    idx_load(0, 0).wait()
    gather(0, 0).start()

    @pl.loop(0, n_chunks, step=2)
    def _(c0):
      for b in range(2):
        c = c0 + b
        nb = 1 - b

        @pl.when(c + 1 < n_chunks)
        def _():
          idx_load(c + 1, nb).wait()
          @pl.when(c >= 1)
          def _():
            put(c - 1, nb).wait()
          gather(c + 1, nb).start()

        gather(c, b).wait()
        put(c, b).start()

        @pl.when(c + 2 < n_chunks)
        def _():
          idx_load(c + 2, b).start()

    put(n_chunks - 2, (n_chunks - 2) % 2).wait()
    put(n_chunks - 1, (n_chunks - 1) % 2).wait()

  return lookup


def kernel(x, weight):
  b, h = x.shape
  d = weight.shape[1]
  xt, weight = jax.lax.optimization_barrier((x.T.astype(jnp.int32), weight))
  out = _build_lookup(b, h, d)(xt, weight)
  return out.reshape(h, b, d).transpose(1, 0, 2)


# speedup vs baseline: 1.1040x; 1.1040x over previous
import functools

import jax
import jax.numpy as jnp
from jax import lax
from jax.experimental import pallas as pl
from jax.experimental.pallas import tpu as pltpu
from jax.experimental.pallas import tpu_sc as plsc

_NC = 2
_NS = 16
_NW = _NC * _NS


@functools.cache
def _build_lookup(n_batch: int, hist: int, d: int):
  chunk = n_batch // _NW
  n_chunks = hist
  assert chunk % 8 == 0 and n_chunks % 2 == 0 and n_chunks >= 4

  mesh = plsc.VectorSubcoreMesh(
      core_axis_name="c", subcore_axis_name="s",
      num_cores=_NC, num_subcores=_NS)

  @functools.partial(
      pl.kernel,
      mesh=mesh,
      out_type=jax.ShapeDtypeStruct((hist * n_batch, d), jnp.float32),
      compiler_params=pltpu.CompilerParams(use_tc_tiling_on_sc=False),
      scratch_types=(
          [pltpu.VMEM((2, chunk), jnp.int32),
           pltpu.VMEM((2, chunk, d), jnp.float32)]
          + [pltpu.SemaphoreType.DMA] * 6
      ),
  )
  def lookup(xt_hbm, w_hbm, out_hbm, idx_v, rows_v, *sems):
    isems = sems[0:2]
    gsems = sems[2:4]
    osems = sems[4:6]
    wid = lax.axis_index("s") * _NC + lax.axis_index("c")
    col0 = wid * chunk

    def idx_load(c, b):
      return pltpu.make_async_copy(
          xt_hbm.at[c, pl.ds(col0, chunk)], idx_v.at[b], isems[b])

    def gather(c, b):
      return pltpu.make_async_copy(
          w_hbm.at[idx_v.at[b]], rows_v.at[b], gsems[b])

    def put(c, b):
      return pltpu.make_async_copy(
          rows_v.at[b],
          out_hbm.at[pl.ds(c * n_batch + col0, chunk)], osems[b])

    idx_load(0, 0).start()
    idx_load(1, 1).start()
    idx_load(0, 0).wait()
    gather(0, 0).start()

    @pl.loop(0, n_chunks, step=2)
    def _(c0):
      for b in range(2):
        c = c0 + b
        nb = 1 - b

        @pl.when(c + 1 < n_chunks)
        def _():
          idx_load(c + 1, nb).wait()
          @pl.when(c >= 1)
          def _():
            put(c - 1, nb).wait()
          gather(c + 1, nb).start()

        gather(c, b).wait()
        put(c, b).start()

        @pl.when(c + 2 < n_chunks)
        def _():
          idx_load(c + 2, b).start()

    put(n_chunks - 2, (n_chunks - 2) % 2).wait()
    put(n_chunks - 1, (n_chunks - 1) % 2).wait()

  return lookup


def kernel(x, weight):
  b, h = x.shape
  d = weight.shape[1]
  out = _build_lookup(b, h, d)(x.T.astype(jnp.int32), weight)
  return out.reshape(h, b, d).transpose(1, 0, 2)
